# baseline (speedup 1.0000x reference)
import functools

import jax
import jax.numpy as jnp
from jax import lax
from jax.experimental import pallas as pl
from jax.experimental.pallas import tpu as pltpu
from jax.experimental.pallas import tpu_sc as plsc

VOCAB = 100000
DIM = 128
BATCH = 4096
HIST = 50

_NC = 2
_NS = 16
_NW = _NC * _NS

_NSPLIT = 4
_CB = BATCH // _NSPLIT
_B_PER_W = _CB // _NW
_FILL = 8
_NFILL = _B_PER_W // _FILL
_NBUF = 2


def _embed_grid(idx_hbm, table_hbm, out_hbm, idx_v, *bufs):
    rows = bufs[:_NBUF]
    gsem = bufs[_NBUF:2 * _NBUF]
    ssem = bufs[2 * _NBUF:]
    w = lax.axis_index("s") * _NC + lax.axis_index("c")
    bbase = w * _B_PER_W
    pltpu.sync_copy(idx_hbm.at[pl.ds(bbase, _B_PER_W)], idx_v)

    def start_fill(b, f):
        for i in range(_FILL):
            pltpu.async_copy(table_hbm.at[idx_v.at[_FILL * f + i]],
                             rows[b].at[i], gsem[b])

    def wait_fill(b):
        pltpu.make_async_copy(out_hbm.at[pl.ds(0, _FILL)], rows[b],
                              gsem[b]).wait()

    def start_store(b, f):
        pltpu.async_copy(rows[b], out_hbm.at[pl.ds(bbase + _FILL * f, _FILL)],
                         ssem[b])

    def wait_store(b):
        pltpu.make_async_copy(rows[b], out_hbm.at[pl.ds(0, _FILL)],
                              ssem[b]).wait()

    for b in range(_NBUF):
        start_fill(b, b)

    def outer(t, carry):
        for b in range(_NBUF):
            f = t * _NBUF + b
            wait_fill(b)
            start_store(b, f)
            wait_store(b)
            start_fill(b, f + _NBUF)
        return carry

    lax.fori_loop(0, _NFILL // _NBUF - 1, outer, 0)
    for b in range(_NBUF):
        f = _NFILL - _NBUF + b
        wait_fill(b)
        start_store(b, f)
    for b in range(_NBUF):
        wait_store(b)


def _make_embed():
    mesh = plsc.VectorSubcoreMesh(core_axis_name="c", subcore_axis_name="s")
    return functools.partial(
        pl.kernel,
        out_type=jax.ShapeDtypeStruct((_CB, HIST, DIM), jnp.float32),
        mesh=mesh,
        scratch_types=(
            [pltpu.VMEM((_B_PER_W, HIST), jnp.int32)]
            + [pltpu.VMEM((_FILL, HIST, DIM), jnp.float32)
               for _ in range(_NBUF)]
            + [pltpu.SemaphoreType.DMA for _ in range(2 * _NBUF)]
        ),
    )(_embed_grid)


@jax.jit
def _embed(idx, table):
    k = _make_embed()
    outs = [k(lax.slice_in_dim(idx, i * _CB, (i + 1) * _CB, axis=0), table)
            for i in range(_NSPLIT)]
    return jnp.concatenate(outs, axis=0)


def kernel(word_vector, weight):
    return _embed(word_vector.astype(jnp.int32), weight)


# speedup vs baseline: 3.2275x; 3.2275x over previous
import functools

import jax
import jax.numpy as jnp
from jax import lax
from jax.experimental import pallas as pl
from jax.experimental.pallas import tpu as pltpu
from jax.experimental.pallas import tpu_sc as plsc

VOCAB = 100000
DIM = 128
BATCH = 4096
HIST = 50

_NC = 2
_NS = 16
_NW = _NC * _NS

_NBC = 16
_NHC = 2
_SPAN = BATCH // _NBC
_HROWS = HIST // _NHC
_IDXC = 128
_GPU = _SPAN // _IDXC
_NBUF = 3


def _embed_grid(idx_hbm, table_hbm, out_hbm, idx_v, *bufs):
    rows = bufs[:_NBUF]
    gsem = bufs[_NBUF:2 * _NBUF]
    ssem = bufs[2 * _NBUF:]
    w = lax.axis_index("s") * _NC + lax.axis_index("c")
    bc = w // _NHC
    hc = w % _NHC
    col0 = bc * _SPAN
    h0 = hc * _HROWS
    for i in range(_GPU):
        pltpu.sync_copy(idx_hbm.at[:, pl.ds(col0 + i * _IDXC, _IDXC)],
                        idx_v.at[:, i])

    def start_gather(b, r):
        for i in range(_GPU):
            pltpu.async_copy(table_hbm.at[idx_v.at[h0 + r, i]],
                             rows[b].at[pl.ds(i * _IDXC, _IDXC)], gsem[b])

    def wait_gather(b):
        pltpu.make_async_copy(table_hbm.at[pl.ds(0, _SPAN)], rows[b],
                              gsem[b]).wait()

    def start_store(b, r):
        pltpu.async_copy(rows[b], out_hbm.at[h0 + r, pl.ds(col0, _SPAN)],
                         ssem[b])

    def wait_store(b):
        pltpu.make_async_copy(rows[b], out_hbm.at[0, pl.ds(col0, _SPAN)],
                              ssem[b]).wait()

    for b in range(_NBUF):
        start_gather(b, b)

    def outer(t, carry):
        for b in range(_NBUF):
            r = t * _NBUF + b
            wait_gather(b)
            start_store(b, r)
            wait_store(b)
            start_gather(b, r + _NBUF)
        return carry

    lax.fori_loop(0, (_HROWS - 1) // _NBUF - 1, outer, 0)
    for r in range(_NBUF * ((_HROWS - 1) // _NBUF - 1), _HROWS):
        b = r % _NBUF
        wait_gather(b)
        start_store(b, r)
        if r + _NBUF < _HROWS:
            wait_store(b)
            start_gather(b, r + _NBUF)
    for b in range(_NBUF):
        wait_store(b)


@jax.jit
def _embed(idx_t, table):
    mesh = plsc.VectorSubcoreMesh(core_axis_name="c", subcore_axis_name="s")
    k = functools.partial(
        pl.kernel,
        out_type=jax.ShapeDtypeStruct((HIST, BATCH, DIM), jnp.float32),
        mesh=mesh,
        scratch_types=(
            [pltpu.VMEM((HIST, _GPU, _IDXC), jnp.int32)]
            + [pltpu.VMEM((_SPAN, DIM), jnp.float32) for _ in range(_NBUF)]
            + [pltpu.SemaphoreType.DMA for _ in range(2 * _NBUF)]
        ),
    )(_embed_grid)
    out = k(idx_t, table)
    return jnp.transpose(out, (1, 0, 2))


def kernel(word_vector, weight):
    return _embed(word_vector.T.astype(jnp.int32), weight)


# speedup vs baseline: 3.3303x; 1.0319x over previous
import functools

import jax
import jax.numpy as jnp
from jax import lax
from jax.experimental import pallas as pl
from jax.experimental.pallas import tpu as pltpu
from jax.experimental.pallas import tpu_sc as plsc

VOCAB = 100000
DIM = 128
BATCH = 4096
HIST = 50

_NC = 2
_NS = 16
_NW = _NC * _NS

_SPAN = BATCH // _NW
_NBUF = 7
_STEADY = (HIST - _NBUF) // _NBUF


def _embed_grid(idx_hbm, table_hbm, out_hbm, idx_v, *bufs):
    rows = bufs[:_NBUF]
    gsem = bufs[_NBUF:2 * _NBUF]
    ssem = bufs[2 * _NBUF:]
    w = lax.axis_index("s") * _NC + lax.axis_index("c")
    col0 = w * _SPAN
    pltpu.sync_copy(idx_hbm.at[:, pl.ds(col0, _SPAN)], idx_v)

    def start_gather(b, h):
        pltpu.async_copy(table_hbm.at[idx_v.at[h]], rows[b], gsem[b])

    def wait_gather(b):
        pltpu.make_async_copy(table_hbm.at[pl.ds(0, _SPAN)], rows[b],
                              gsem[b]).wait()

    def start_store(b, h):
        pltpu.async_copy(rows[b], out_hbm.at[h, pl.ds(col0, _SPAN)], ssem[b])

    def wait_store(b):
        pltpu.make_async_copy(rows[b], out_hbm.at[0, pl.ds(col0, _SPAN)],
                              ssem[b]).wait()

    for b in range(_NBUF):
        start_gather(b, b)

    def outer(t, carry):
        for b in range(_NBUF):
            h = t * _NBUF + b
            wait_gather(b)
            start_store(b, h)
            wait_store(b)
            start_gather(b, h + _NBUF)
        return carry

    lax.fori_loop(0, _STEADY, outer, 0)
    for h in range(_STEADY * _NBUF, HIST):
        b = h % _NBUF
        wait_gather(b)
        start_store(b, h)
        if h + _NBUF < HIST:
            wait_store(b)
            start_gather(b, h + _NBUF)
    for b in range(_NBUF):
        wait_store(b)


@jax.jit
def _embed(idx_t, table):
    mesh = plsc.VectorSubcoreMesh(core_axis_name="c", subcore_axis_name="s")
    k = functools.partial(
        pl.kernel,
        out_type=jax.ShapeDtypeStruct((HIST, BATCH, DIM), jnp.float32),
        mesh=mesh,
        scratch_types=(
            [pltpu.VMEM((HIST, _SPAN), jnp.int32)]
            + [pltpu.VMEM((_SPAN, DIM), jnp.float32) for _ in range(_NBUF)]
            + [pltpu.SemaphoreType.DMA for _ in range(2 * _NBUF)]
        ),
    )(_embed_grid)
    out = k(idx_t, table)
    return jnp.transpose(out, (1, 0, 2))


def kernel(word_vector, weight):
    return _embed(word_vector.T.astype(jnp.int32), weight)
